# baseline (speedup 1.0000x reference)
import functools
import math

import jax
import jax.numpy as jnp
from jax import lax
from jax.experimental import pallas as pl
from jax.experimental.pallas import tpu as pltpu

BSZ = 128
N_TOP = 2
TEMPERATURE = 1.0
MASK_VALUE = -jnp.finfo(jnp.float32).max
ROWS = 4


def _sortnet_picks(qr_ref, kr_ref, r, *, buckets, bsz, dh, t, tb):
    qb = qr_ref[r * t:(r + 1) * t, :].reshape(buckets, bsz, dh)
    kb = kr_ref[r * tb + bsz:(r + 1) * tb, :].reshape(buckets, bsz, dh)

    r_i = lax.broadcasted_iota(jnp.int32, (buckets, buckets), 0)
    c_i = lax.broadcasted_iota(jnp.int32, (buckets, buckets), 1)
    tril_strict = (c_i < r_i).astype(jnp.float32)

    qs = qb.sum(axis=1)
    q_prefix = jnp.dot(tril_strict, qs, preferred_element_type=jnp.float32,
                       precision=lax.Precision.HIGHEST)
    q_first = qb[:, 0, :]
    pos = lax.broadcasted_iota(jnp.int32, (buckets, dh), 0).astype(jnp.float32)
    sq = (q_prefix + q_first) / (pos * bsz + 1.0)

    ks = kb.sum(axis=1)
    k_prefix = jnp.dot(tril_strict, ks, preferred_element_type=jnp.float32,
                       precision=lax.Precision.HIGHEST)
    jj = lax.broadcasted_iota(jnp.int32, (buckets, bsz), 0).astype(jnp.float32)
    rr = lax.broadcasted_iota(jnp.int32, (buckets, bsz), 1).astype(jnp.float32)
    inv_t = 1.0 / (jj * bsz + rr + 1.0)
    ge_r = lax.broadcasted_iota(jnp.int32, (bsz, bsz), 0)
    ge_c = lax.broadcasted_iota(jnp.int32, (bsz, bsz), 1)
    suffix = (ge_r >= ge_c).astype(jnp.float32)
    ws = jnp.dot(inv_t, suffix, preferred_element_type=jnp.float32,
                 precision=lax.Precision.HIGHEST)
    w_total = ws[:, 0:1]
    weighted = lax.dot_general(
        ws, kb, (((1,), (1,)), ((0,), (0,))),
        preferred_element_type=jnp.float32,
        precision=lax.Precision.HIGHEST)
    sk = k_prefix * w_total + weighted

    scores = jax.lax.dot_general(
        sq, sk, (((1,), (1,)), ((), ())),
        preferred_element_type=jnp.float32) * (dh ** -0.5)
    cols = buckets + N_TOP
    full = jnp.concatenate(
        [jnp.zeros((buckets, N_TOP), jnp.float32), scores], axis=1)
    mrow = lax.broadcasted_iota(jnp.int32, (buckets, cols), 0)
    mcol = lax.broadcasted_iota(jnp.int32, (buckets, cols), 1)
    masked = (mcol >= N_TOP) & ((mcol - N_TOP) >= mrow)
    x = jnp.where(masked, MASK_VALUE, full)

    idxs = []
    vals = []
    for n in range(N_TOP):
        sm = jax.nn.softmax(x / TEMPERATURE, axis=-1)
        idx_n = jnp.argmax(sm, axis=-1).astype(jnp.int32)
        val_n = jnp.max(sm, axis=-1)
        idxs.append(idx_n)
        vals.append(val_n)
        if n != N_TOP - 1:
            x = jnp.where(mcol == idx_n[:, None], -jnp.inf, x)
    return idxs, vals


def _fused_kernel(q_ref, k_ref, v_ref, nk_ref, nv_ref,
                  o_ref, qr_ref, kr_ref, vr_ref, *, buckets, bsz, dh, hh):
    i = pl.program_id(0)
    lg2e = (dh ** -0.5) * math.log2(math.e)
    t = buckets * bsz
    tb = t + bsz
    sh = bsz - 1

    for r in range(ROWS):
        row = ROWS * i + r
        is_rolled = (row % (2 * hh)) >= hh
        kr_ref[r * tb:r * tb + bsz, :] = jnp.broadcast_to(
            nk_ref[r], (bsz, dh))
        vr_ref[r * tb:r * tb + bsz, :] = jnp.broadcast_to(
            nv_ref[r], (bsz, dh))

        @pl.when(is_rolled)
        def _(r=r):
            q_row = q_ref[r]
            k_row = k_ref[r]
            v_row = v_ref[r]
            qr_ref[r * t:(r + 1) * t, :] = jnp.concatenate(
                [q_row[sh:], q_row[:sh]], axis=0)
            kr_ref[r * tb + bsz:(r + 1) * tb, :] = jnp.concatenate(
                [k_row[sh:], k_row[:sh]], axis=0)
            vr_ref[r * tb + bsz:(r + 1) * tb, :] = jnp.concatenate(
                [v_row[sh:], v_row[:sh]], axis=0)

        @pl.when(jnp.logical_not(is_rolled))
        def _(r=r):
            qr_ref[r * t:(r + 1) * t, :] = q_ref[r]
            kr_ref[r * tb + bsz:(r + 1) * tb, :] = k_ref[r]
            vr_ref[r * tb + bsz:(r + 1) * tb, :] = v_ref[r]

    ri = lax.broadcasted_iota(jnp.int32, (bsz, bsz), 0)
    ci = lax.broadcasted_iota(jnp.int32, (bsz, bsz), 1)
    causal = ci > ri

    for r in range(ROWS):
        row = ROWS * i + r
        is_rolled = (row % (2 * hh)) >= hh
        (idx1, idx2), (vals1, vals2) = _sortnet_picks(
            qr_ref, kr_ref, r, buckets=buckets, bsz=bsz, dh=dh, t=t, tb=tb)

        outs = []
        for u in range(buckets):
            q_u = qr_ref[r * t + u * bsz:r * t + (u + 1) * bsz, :]
            i1 = idx1[u]
            i2 = idx2[u]
            val1 = vals1[u]
            val2 = vals2[u]
            s1 = r * tb + jnp.where(i1 < N_TOP, 0, (i1 - 1) * bsz)
            s2 = r * tb + jnp.where(i2 < N_TOP, 0, (i2 - 1) * bsz)
            k1 = kr_ref[pl.ds(s1, bsz), :]
            k2 = kr_ref[pl.ds(s2, bsz), :]
            v1 = vr_ref[pl.ds(s1, bsz), :]
            v2 = vr_ref[pl.ds(s2, bsz), :]
            kl = kr_ref[r * tb + (u + 1) * bsz:r * tb + (u + 2) * bsz, :]
            vl = vr_ref[r * tb + (u + 1) * bsz:r * tb + (u + 2) * bsz, :]

            qh = q_u.astype(jnp.bfloat16)
            l1 = jnp.dot(qh, k1.astype(jnp.bfloat16).T,
                         preferred_element_type=jnp.float32) * (lg2e * val1)
            l2 = jnp.dot(qh, k2.astype(jnp.bfloat16).T,
                         preferred_element_type=jnp.float32) * (lg2e * val2)
            l3 = jnp.dot(qh, kl.astype(jnp.bfloat16).T,
                         preferred_element_type=jnp.float32) * lg2e
            l3 = jnp.where(causal, MASK_VALUE, l3)

            p1 = jnp.exp2(l1)
            p2 = jnp.exp2(l2)
            p3 = jnp.exp2(l3)
            denom = (p1 + p2 + p3).sum(axis=-1)[:, None]
            acc = (jnp.dot(p1.astype(jnp.bfloat16), v1.astype(jnp.bfloat16),
                           preferred_element_type=jnp.float32) * val1
                   + jnp.dot(p2.astype(jnp.bfloat16), v2.astype(jnp.bfloat16),
                             preferred_element_type=jnp.float32) * val2
                   + jnp.dot(p3.astype(jnp.bfloat16), vl.astype(jnp.bfloat16),
                             preferred_element_type=jnp.float32))
            outs.append(acc / denom)

        out_row = jnp.concatenate(outs, axis=0)

        @pl.when(is_rolled)
        def _(r=r, out_row=out_row):
            o_ref[r] = jnp.concatenate(
                [out_row[t - sh:], out_row[:t - sh]], axis=0)

        @pl.when(jnp.logical_not(is_rolled))
        def _(r=r, out_row=out_row):
            o_ref[r] = out_row


def _sinkhorn_attention(q, k, v, null_keys, null_values):
    b, h, t, dh = q.shape
    bsz = BSZ
    bh = b * h
    hh = h // 2
    buckets = t // bsz

    q = q.reshape(bh, t, dh)
    k = k.reshape(bh, t, dh)
    v = v.reshape(bh, t, dh)
    nk = jnp.broadcast_to(null_keys[None, :, 0, :],
                          (b, h, dh)).reshape(bh, 1, dh)
    nv = jnp.broadcast_to(null_values[None, :, 0, :],
                          (b, h, dh)).reshape(bh, 1, dh)

    row_spec = pl.BlockSpec((ROWS, t, dh), lambda i: (i, 0, 0))
    null_spec = pl.BlockSpec((ROWS, 1, dh), lambda i: (i, 0, 0))

    out = pl.pallas_call(
        functools.partial(_fused_kernel,
                          buckets=buckets, bsz=bsz, dh=dh, hh=hh),
        grid=(bh // ROWS,),
        in_specs=[row_spec, row_spec, row_spec, null_spec, null_spec],
        out_specs=row_spec,
        out_shape=jax.ShapeDtypeStruct((bh, t, dh), jnp.float32),
        scratch_shapes=[
            pltpu.VMEM((ROWS * t, dh), jnp.float32),
            pltpu.VMEM((ROWS * (t + bsz), dh), jnp.float32),
            pltpu.VMEM((ROWS * (t + bsz), dh), jnp.float32),
        ],
    )(q, k, v, nk, nv)

    return out.reshape(b, h, t, dh)


def kernel(q, k, v, null_keys, null_values):
    return _sinkhorn_attention(q, k, v, null_keys, null_values)


# speedup vs baseline: 1.3221x; 1.3221x over previous
import functools
import math

import jax
import jax.numpy as jnp
from jax import lax
from jax.experimental import pallas as pl
from jax.experimental.pallas import tpu as pltpu

BSZ = 128
N_TOP = 2
TEMPERATURE = 1.0
MASK_VALUE = -jnp.finfo(jnp.float32).max
ROWS = 4


def _sortnet_picks(qr_ref, kr_ref, r, *, buckets, bsz, dh, t, tb):
    qb = qr_ref[r * t:(r + 1) * t, :].reshape(buckets, bsz, dh)
    kb = kr_ref[r * tb + bsz:(r + 1) * tb, :].reshape(buckets, bsz, dh)

    r_i = lax.broadcasted_iota(jnp.int32, (buckets, buckets), 0)
    c_i = lax.broadcasted_iota(jnp.int32, (buckets, buckets), 1)
    tril_strict = (c_i < r_i).astype(jnp.float32)

    qs = qb.sum(axis=1)
    q_prefix = jnp.dot(tril_strict, qs, preferred_element_type=jnp.float32,
                       precision=lax.Precision.HIGHEST)
    q_first = qb[:, 0, :]
    pos = lax.broadcasted_iota(jnp.int32, (buckets, dh), 0).astype(jnp.float32)
    sq = (q_prefix + q_first) / (pos * bsz + 1.0)

    ks = kb.sum(axis=1)
    k_prefix = jnp.dot(tril_strict, ks, preferred_element_type=jnp.float32,
                       precision=lax.Precision.HIGHEST)
    jj = lax.broadcasted_iota(jnp.int32, (buckets, bsz), 0).astype(jnp.float32)
    rr = lax.broadcasted_iota(jnp.int32, (buckets, bsz), 1).astype(jnp.float32)
    inv_t = 1.0 / (jj * bsz + rr + 1.0)
    ge_r = lax.broadcasted_iota(jnp.int32, (bsz, bsz), 0)
    ge_c = lax.broadcasted_iota(jnp.int32, (bsz, bsz), 1)
    suffix = (ge_r >= ge_c).astype(jnp.float32)
    ws = jnp.dot(inv_t, suffix, preferred_element_type=jnp.float32,
                 precision=lax.Precision.HIGHEST)
    w_total = ws[:, 0:1]
    weighted = (kb * ws[:, :, None]).sum(axis=1)
    sk = k_prefix * w_total + weighted

    scores = jax.lax.dot_general(
        sq, sk, (((1,), (1,)), ((), ())),
        preferred_element_type=jnp.float32) * (dh ** -0.5)
    cols = buckets + N_TOP
    full = jnp.concatenate(
        [jnp.zeros((buckets, N_TOP), jnp.float32), scores], axis=1)
    mrow = lax.broadcasted_iota(jnp.int32, (buckets, cols), 0)
    mcol = lax.broadcasted_iota(jnp.int32, (buckets, cols), 1)
    masked = (mcol >= N_TOP) & ((mcol - N_TOP) >= mrow)
    x = jnp.where(masked, MASK_VALUE, full)

    idxs = []
    vals = []
    for n in range(N_TOP):
        sm = jax.nn.softmax(x / TEMPERATURE, axis=-1)
        idx_n = jnp.argmax(sm, axis=-1).astype(jnp.int32)
        val_n = jnp.max(sm, axis=-1)
        idxs.append(idx_n)
        vals.append(val_n)
        if n != N_TOP - 1:
            x = jnp.where(mcol == idx_n[:, None], -jnp.inf, x)
    return idxs, vals


def _fused_kernel(q_ref, k_ref, v_ref, nk_ref, nv_ref,
                  o_ref, qr_ref, kr_ref, vr_ref, *, buckets, bsz, dh, hh):
    i = pl.program_id(0)
    lg2e = (dh ** -0.5) * math.log2(math.e)
    t = buckets * bsz
    tb = t + bsz
    sh = bsz - 1

    for r in range(ROWS):
        row = ROWS * i + r
        is_rolled = (row % (2 * hh)) >= hh
        kr_ref[r * tb:r * tb + bsz, :] = jnp.broadcast_to(
            nk_ref[r], (bsz, dh))
        vr_ref[r * tb:r * tb + bsz, :] = jnp.broadcast_to(
            nv_ref[r], (bsz, dh))

        @pl.when(is_rolled)
        def _(r=r):
            q_row = q_ref[r]
            k_row = k_ref[r]
            v_row = v_ref[r]
            qr_ref[r * t:(r + 1) * t, :] = jnp.concatenate(
                [q_row[sh:], q_row[:sh]], axis=0)
            kr_ref[r * tb + bsz:(r + 1) * tb, :] = jnp.concatenate(
                [k_row[sh:], k_row[:sh]], axis=0)
            vr_ref[r * tb + bsz:(r + 1) * tb, :] = jnp.concatenate(
                [v_row[sh:], v_row[:sh]], axis=0)

        @pl.when(jnp.logical_not(is_rolled))
        def _(r=r):
            qr_ref[r * t:(r + 1) * t, :] = q_ref[r]
            kr_ref[r * tb + bsz:(r + 1) * tb, :] = k_ref[r]
            vr_ref[r * tb + bsz:(r + 1) * tb, :] = v_ref[r]

    ri = lax.broadcasted_iota(jnp.int32, (bsz, bsz), 0)
    ci = lax.broadcasted_iota(jnp.int32, (bsz, bsz), 1)
    causal = ci > ri

    picks = [_sortnet_picks(qr_ref, kr_ref, r, buckets=buckets, bsz=bsz,
                            dh=dh, t=t, tb=tb) for r in range(ROWS)]

    for r in range(ROWS):
        row = ROWS * i + r
        is_rolled = (row % (2 * hh)) >= hh
        (idx1, idx2), (vals1, vals2) = picks[r]
        outs = []
        for u in range(buckets):
            q_u = qr_ref[r * t + u * bsz:r * t + (u + 1) * bsz, :]
            i1 = idx1[u]
            i2 = idx2[u]
            val1 = vals1[u]
            val2 = vals2[u]
            s1 = r * tb + jnp.where(i1 < N_TOP, 0, (i1 - 1) * bsz)
            s2 = r * tb + jnp.where(i2 < N_TOP, 0, (i2 - 1) * bsz)
            k1 = kr_ref[pl.ds(s1, bsz), :]
            k2 = kr_ref[pl.ds(s2, bsz), :]
            v1 = vr_ref[pl.ds(s1, bsz), :]
            v2 = vr_ref[pl.ds(s2, bsz), :]
            kl = kr_ref[r * tb + (u + 1) * bsz:r * tb + (u + 2) * bsz, :]
            vl = vr_ref[r * tb + (u + 1) * bsz:r * tb + (u + 2) * bsz, :]

            qh = q_u.astype(jnp.bfloat16)
            l1 = jnp.dot(qh, k1.astype(jnp.bfloat16).T,
                         preferred_element_type=jnp.float32) * (lg2e * val1)
            l2 = jnp.dot(qh, k2.astype(jnp.bfloat16).T,
                         preferred_element_type=jnp.float32) * (lg2e * val2)
            l3 = jnp.dot(qh, kl.astype(jnp.bfloat16).T,
                         preferred_element_type=jnp.float32) * lg2e
            l3 = jnp.where(causal, MASK_VALUE, l3)

            p1 = jnp.exp2(l1)
            p2 = jnp.exp2(l2)
            p3 = jnp.exp2(l3)
            denom = (p1 + p2 + p3).sum(axis=-1)[:, None]
            acc = (jnp.dot(p1.astype(jnp.bfloat16), v1.astype(jnp.bfloat16),
                           preferred_element_type=jnp.float32) * val1
                   + jnp.dot(p2.astype(jnp.bfloat16), v2.astype(jnp.bfloat16),
                             preferred_element_type=jnp.float32) * val2
                   + jnp.dot(p3.astype(jnp.bfloat16), vl.astype(jnp.bfloat16),
                             preferred_element_type=jnp.float32))
            outs.append(acc / denom)

        out_row = jnp.concatenate(outs, axis=0)

        @pl.when(is_rolled)
        def _(r=r, out_row=out_row):
            o_ref[r] = jnp.concatenate(
                [out_row[t - sh:], out_row[:t - sh]], axis=0)

        @pl.when(jnp.logical_not(is_rolled))
        def _(r=r, out_row=out_row):
            o_ref[r] = out_row


def _sinkhorn_attention(q, k, v, null_keys, null_values):
    b, h, t, dh = q.shape
    bsz = BSZ
    bh = b * h
    hh = h // 2
    buckets = t // bsz

    q = q.reshape(bh, t, dh)
    k = k.reshape(bh, t, dh)
    v = v.reshape(bh, t, dh)
    nk = jnp.broadcast_to(null_keys[None, :, 0, :],
                          (b, h, dh)).reshape(bh, 1, dh)
    nv = jnp.broadcast_to(null_values[None, :, 0, :],
                          (b, h, dh)).reshape(bh, 1, dh)

    row_spec = pl.BlockSpec((ROWS, t, dh), lambda i: (i, 0, 0))
    null_spec = pl.BlockSpec((ROWS, 1, dh), lambda i: (i, 0, 0))

    out = pl.pallas_call(
        functools.partial(_fused_kernel,
                          buckets=buckets, bsz=bsz, dh=dh, hh=hh),
        grid=(bh // ROWS,),
        in_specs=[row_spec, row_spec, row_spec, null_spec, null_spec],
        out_specs=row_spec,
        out_shape=jax.ShapeDtypeStruct((bh, t, dh), jnp.float32),
        scratch_shapes=[
            pltpu.VMEM((ROWS * t, dh), jnp.float32),
            pltpu.VMEM((ROWS * (t + bsz), dh), jnp.float32),
            pltpu.VMEM((ROWS * (t + bsz), dh), jnp.float32),
        ],
    )(q, k, v, nk, nv)

    return out.reshape(b, h, t, dh)


def kernel(q, k, v, null_keys, null_values):
    return _sinkhorn_attention(q, k, v, null_keys, null_values)
